# Initial kernel scaffold; baseline (speedup 1.0000x reference)
import functools

import jax
import jax.numpy as jnp
from jax import lax
from jax.experimental import pallas as pl
from jax.experimental.pallas import tpu as pltpu
from jax.experimental.pallas import tpu_sc as plsc

N = 10000
E = 320000
D = 128

NC = 2
NS = 16
NW = NC * NS

NPAD = 10240
LDEG = 2 * NPAD
EPW = E // NW
DEG_ROWS = EPW // 16
RED = LDEG // NS

C = 80
CH = EPW // C
ROWS_PER_TILE = N // NS

_mesh = plsc.VectorSubcoreMesh(core_axis_name="c", subcore_axis_name="s")


def _deg_body(src16, dst16, degs, vsrc, vdst, ldeg, rbuf, obuf, slots):
    c = lax.axis_index("c")
    s = lax.axis_index("s")
    wid = c * NS + s
    pltpu.sync_copy(src16.at[pl.ds(wid * DEG_ROWS, DEG_ROWS)], vsrc)
    pltpu.sync_copy(dst16.at[pl.ds(wid * DEG_ROWS, DEG_ROWS)], vdst)

    zeros = jnp.zeros((16,), jnp.float32)
    ones = jnp.ones((16,), jnp.float32)

    def zbody(i, _):
        ldeg[pl.ds(i * 16, 16)] = zeros
        return 0
    lax.fori_loop(0, LDEG // 16, zbody, 0)

    def hbody(j, _):
        plsc.addupdate_scatter(ldeg, [vsrc[j]], ones)
        plsc.addupdate_scatter(ldeg, [vdst[j] + NPAD], ones)
        return 0
    lax.fori_loop(0, DEG_ROWS, hbody, 0)

    pltpu.sync_copy(ldeg, slots.at[s])
    plsc.subcore_barrier()
    pltpu.sync_copy(slots.at[:, pl.ds(s * RED, RED)], rbuf)

    def rbody(v, _):
        acc = rbuf[0, pl.ds(v * 16, 16)]
        for t in range(1, NS):
            acc = acc + rbuf[t, pl.ds(v * 16, 16)]
        obuf[pl.ds(v * 16, 16)] = acc
        return 0
    lax.fori_loop(0, RED // 16, rbody, 0)
    pltpu.sync_copy(obuf, degs.at[c, pl.ds(s * RED, RED)])


_deg_kernel = functools.partial(
    pl.kernel,
    out_type=jax.ShapeDtypeStruct((NC, LDEG), jnp.float32),
    mesh=_mesh,
    scratch_types=[
        pltpu.VMEM((DEG_ROWS, 16), jnp.int32),
        pltpu.VMEM((DEG_ROWS, 16), jnp.int32),
        pltpu.VMEM((LDEG,), jnp.float32),
        pltpu.VMEM((NS, RED), jnp.float32),
        pltpu.VMEM((RED,), jnp.float32),
        pltpu.VMEM_SHARED((NS, LDEG), jnp.float32),
    ],
)(_deg_body)


def _edge_body(h, src80, dst80, part, visrc, vidst, rowsA, rowsB, zbuf, agg,
               semA, semB):
    c = lax.axis_index("c")
    s = lax.axis_index("s")
    wid = c * NS + s
    pltpu.sync_copy(src80.at[pl.ds(wid * CH, CH)], visrc)
    pltpu.sync_copy(dst80.at[pl.ds(wid * CH, CH)], vidst)

    zeros = jnp.zeros((16,), jnp.float32)

    def zbody(i, _):
        for k in range(D // 16):
            zbuf[i, pl.ds(k * 16, 16)] = zeros
        return 0
    lax.fori_loop(0, 125, zbody, 0)
    for k in range(ROWS_PER_TILE // 125):
        pltpu.sync_copy(zbuf, agg.at[pl.ds(s * ROWS_PER_TILE + k * 125, 125)])
    plsc.subcore_barrier()

    pltpu.async_copy(h.at[visrc.at[0]], rowsA, semA)

    def body(k, _):
        j = 2 * k
        pltpu.async_copy(h.at[visrc.at[j + 1]], rowsB, semB)
        pltpu.make_async_copy(h.at[visrc.at[j]], rowsA, semA).wait()
        pltpu.sync_copy(rowsA, agg.at[vidst.at[j]], add=True)
        pltpu.async_copy(h.at[visrc.at[j + 2]], rowsA, semA)
        pltpu.make_async_copy(h.at[visrc.at[j + 1]], rowsB, semB).wait()
        pltpu.sync_copy(rowsB, agg.at[vidst.at[j + 1]], add=True)
        return 0
    lax.fori_loop(0, (CH - 1) // 2, body, 0)
    last = CH - 1
    pltpu.make_async_copy(h.at[visrc.at[last]], rowsA, semA).wait()
    pltpu.sync_copy(rowsA, agg.at[vidst.at[last]], add=True)

    plsc.subcore_barrier()
    pltpu.sync_copy(agg.at[pl.ds(s * ROWS_PER_TILE, ROWS_PER_TILE)],
                    part.at[c, pl.ds(s * ROWS_PER_TILE, ROWS_PER_TILE)])


_edge_kernel = functools.partial(
    pl.kernel,
    out_type=jax.ShapeDtypeStruct((NC, N, D), jnp.float32),
    mesh=_mesh,
    scratch_types=[
        pltpu.VMEM((CH, C), jnp.int32),
        pltpu.VMEM((CH, C), jnp.int32),
        pltpu.VMEM((C, D), jnp.float32),
        pltpu.VMEM((C, D), jnp.float32),
        pltpu.VMEM((125, D), jnp.float32),
        pltpu.VMEM_SHARED((N, D), jnp.float32),
        pltpu.SemaphoreType.DMA,
        pltpu.SemaphoreType.DMA,
    ],
)(_edge_body)


BR = 400
GRID = N // BR


def _mm1_body(x_ref, w_ref, dsrc_ref, o_ref):
    nsrc = lax.rsqrt(jnp.maximum(dsrc_ref[...], 1.0))
    o_ref[...] = jnp.dot(x_ref[...], w_ref[...],
                         preferred_element_type=jnp.float32) * nsrc


def _mid_body(p_ref, ddst_ref, b1_ref, w_ref, dsrc_ref, o_ref):
    ndst = lax.rsqrt(jnp.maximum(ddst_ref[...], 1.0))
    nsrc = lax.rsqrt(jnp.maximum(dsrc_ref[...], 1.0))
    agg = (p_ref[0] + p_ref[1]) * ndst
    hmid = jnp.maximum(agg + b1_ref[...], 0.0)
    o_ref[...] = jnp.dot(hmid, w_ref[...],
                         preferred_element_type=jnp.float32) * nsrc


def _fin_body(p_ref, ddst_ref, b2_ref, g_ref, bt_ref, o_ref):
    ndst = lax.rsqrt(jnp.maximum(ddst_ref[...], 1.0))
    hval = (p_ref[0] + p_ref[1]) * ndst + b2_ref[...]
    mu = jnp.mean(hval, axis=-1, keepdims=True)
    dvar = hval - mu
    var = jnp.mean(dvar * dvar, axis=-1, keepdims=True)
    o_ref[...] = dvar * lax.rsqrt(var + 1e-5) * g_ref[...] + bt_ref[...]


_row_spec = pl.BlockSpec((BR, D), lambda i: (i, 0))
_col_spec = pl.BlockSpec((BR, 1), lambda i: (i, 0))
_full_spec = pl.BlockSpec((1, D), lambda i: (0, 0))
_w_spec = pl.BlockSpec((D, D), lambda i: (0, 0))
_p_spec = pl.BlockSpec((NC, BR, D), lambda i: (0, i, 0))
_out_shape = jax.ShapeDtypeStruct((N, D), jnp.float32)

_mm1 = pl.pallas_call(
    _mm1_body, grid=(GRID,),
    in_specs=[_row_spec, _w_spec, _col_spec],
    out_specs=_row_spec, out_shape=_out_shape)

_mid = pl.pallas_call(
    _mid_body, grid=(GRID,),
    in_specs=[_p_spec, _col_spec, _full_spec, _w_spec, _col_spec],
    out_specs=_row_spec, out_shape=_out_shape)

_fin = pl.pallas_call(
    _fin_body, grid=(GRID,),
    in_specs=[_p_spec, _col_spec, _full_spec, _full_spec, _full_spec],
    out_specs=_row_spec, out_shape=_out_shape)


def kernel(x, edge_index, W1, b1, W2, b2, ln_gamma, ln_beta):
    src = edge_index[0]
    dst = edge_index[1]
    src16 = src.reshape(E // 16, 16)
    dst16 = dst.reshape(E // 16, 16)
    src80 = src.reshape(E // C, C)
    dst80 = dst.reshape(E // C, C)

    degs = _deg_kernel(src16, dst16)
    dsrc = (degs[0, :N] + degs[1, :N]).reshape(N, 1)
    ddst = (degs[0, NPAD:NPAD + N] + degs[1, NPAD:NPAD + N]).reshape(N, 1)

    b1r = b1.reshape(1, D)
    b2r = b2.reshape(1, D)
    gr = ln_gamma.reshape(1, D)
    btr = ln_beta.reshape(1, D)

    h1 = _mm1(x, W1, dsrc)
    p1 = _edge_kernel(h1, src80, dst80)
    h2 = _mid(p1, ddst, b1r, W2, dsrc)
    p2 = _edge_kernel(h2, src80, dst80)
    return _fin(p2, ddst, b2r, gr, btr)


# speedup vs baseline: 11.0979x; 11.0979x over previous
import functools

import jax
import jax.numpy as jnp
from jax import lax
from jax.experimental import pallas as pl
from jax.experimental.pallas import tpu as pltpu
from jax.experimental.pallas import tpu_sc as plsc

N = 10000
E = 320000
D = 128

NC = 2
NS = 16
NW = NC * NS

NPAD = 10240
LDEG = 2 * NPAD
EPW = E // NW
DEG_ROWS = EPW // 16
RED = LDEG // NS

C = 80
CH = EPW // C
SPAN = 624

_mesh = plsc.VectorSubcoreMesh(core_axis_name="c", subcore_axis_name="s")


def _deg_body(src16, dst16, degs, vsrc, vdst, ldeg, rbuf, obuf, slots):
    c = lax.axis_index("c")
    s = lax.axis_index("s")
    wid = c * NS + s
    pltpu.sync_copy(src16.at[wid], vsrc)
    pltpu.sync_copy(dst16.at[wid], vdst)

    zeros = jnp.zeros((16,), jnp.float32)
    ones = jnp.ones((16,), jnp.float32)

    def zbody(i, _):
        ldeg[pl.ds(i * 16, 16)] = zeros
        return 0
    lax.fori_loop(0, LDEG // 16, zbody, 0)

    def hbody(j, _):
        plsc.addupdate_scatter(ldeg, [vsrc[j]], ones)
        plsc.addupdate_scatter(ldeg, [vdst[j] + NPAD], ones)
        return 0
    lax.fori_loop(0, DEG_ROWS, hbody, 0)

    pltpu.sync_copy(ldeg, slots.at[pl.ds(s * LDEG, LDEG)])
    plsc.subcore_barrier()
    for t in range(NS):
        pltpu.sync_copy(slots.at[pl.ds(t * LDEG + s * RED, RED)],
                        rbuf.at[pl.ds(t * RED, RED)])

    def rbody(v, _):
        acc = rbuf[pl.ds(v * 16, 16)]
        for t in range(1, NS):
            acc = acc + rbuf[pl.ds(t * RED + v * 16, 16)]
        obuf[pl.ds(v * 16, 16)] = acc
        return 0
    lax.fori_loop(0, RED // 16, rbody, 0)
    pltpu.sync_copy(obuf, degs.at[c, 0, pl.ds(s * RED, RED)])


_deg_kernel = functools.partial(
    pl.kernel,
    out_type=jax.ShapeDtypeStruct((NC, 1, LDEG), jnp.float32),
    mesh=_mesh,
    scratch_types=[
        pltpu.VMEM((DEG_ROWS, 16), jnp.int32),
        pltpu.VMEM((DEG_ROWS, 16), jnp.int32),
        pltpu.VMEM((LDEG,), jnp.float32),
        pltpu.VMEM((NS * RED,), jnp.float32),
        pltpu.VMEM((RED,), jnp.float32),
        pltpu.VMEM_SHARED((NS * LDEG,), jnp.float32),
    ],
    compiler_params=pltpu.CompilerParams(
        needs_layout_passes=False, use_tc_tiling_on_sc=False),
)(_deg_body)


def _edge_body(h, src80, dst80, part, visrc, vidst, rowsA, rowsB, zbuf, agg,
               semA, semB):
    c = lax.axis_index("c")
    s = lax.axis_index("s")
    wid = c * NS + s
    pltpu.sync_copy(src80.at[wid], visrc)
    pltpu.sync_copy(dst80.at[wid], vidst)

    zeros = jnp.zeros((16,), jnp.float32)

    def zvbody(i, _):
        for k in range(D // 16):
            zbuf[i, pl.ds(k * 16, 16)] = zeros
        return 0
    lax.fori_loop(0, 16, zvbody, 0)

    def zbody(k, _):
        pltpu.sync_copy(zbuf, agg.at[pl.ds(s * SPAN + k * 16, 16)])
        return 0
    lax.fori_loop(0, SPAN // 16, zbody, 0)

    @pl.when(s == NS - 1)
    def _():
        pltpu.sync_copy(zbuf, agg.at[pl.ds(NS * SPAN, 16)])

    plsc.subcore_barrier()

    pltpu.async_copy(h.at[visrc.at[0]], rowsA, semA)

    def body(k, _):
        j = 2 * k
        pltpu.async_copy(h.at[visrc.at[j + 1]], rowsB, semB)
        pltpu.make_async_copy(h.at[visrc.at[j]], rowsA, semA).wait()
        pltpu.sync_copy(rowsA, agg.at[vidst.at[j]], add=True)
        pltpu.async_copy(h.at[visrc.at[j + 2]], rowsA, semA)
        pltpu.make_async_copy(h.at[visrc.at[j + 1]], rowsB, semB).wait()
        pltpu.sync_copy(rowsB, agg.at[vidst.at[j + 1]], add=True)
        return 0
    lax.fori_loop(0, (CH - 1) // 2, body, 0)
    last = CH - 1
    pltpu.make_async_copy(h.at[visrc.at[last]], rowsA, semA).wait()
    pltpu.sync_copy(rowsA, agg.at[vidst.at[last]], add=True)

    plsc.subcore_barrier()
    pltpu.sync_copy(agg.at[pl.ds(s * SPAN, SPAN)],
                    part.at[c, pl.ds(s * SPAN, SPAN)])

    @pl.when(s == NS - 1)
    def _():
        pltpu.sync_copy(agg.at[pl.ds(NS * SPAN, 16)],
                        part.at[c, pl.ds(NS * SPAN, 16)])


_edge_kernel = functools.partial(
    pl.kernel,
    out_type=jax.ShapeDtypeStruct((NC, N, D), jnp.float32),
    mesh=_mesh,
    scratch_types=[
        pltpu.VMEM((CH, C), jnp.int32),
        pltpu.VMEM((CH, C), jnp.int32),
        pltpu.VMEM((C, D), jnp.float32),
        pltpu.VMEM((C, D), jnp.float32),
        pltpu.VMEM((16, D), jnp.float32),
        pltpu.VMEM_SHARED((N, D), jnp.float32),
        pltpu.SemaphoreType.DMA,
        pltpu.SemaphoreType.DMA,
    ],
    compiler_params=pltpu.CompilerParams(
        needs_layout_passes=False, use_tc_tiling_on_sc=False),
)(_edge_body)


BR = 400
GRID = N // BR


def _mm1_body(x_ref, w_ref, dsrc_ref, o_ref):
    nsrc = lax.rsqrt(jnp.maximum(dsrc_ref[...], 1.0))
    o_ref[...] = jnp.dot(x_ref[...], w_ref[...],
                         preferred_element_type=jnp.float32) * nsrc


def _post_body(p_ref, ddst_ref, b_ref, w_ref, dsrc_ref, g_ref, bt_ref,
               flag_ref, o_ref):
    ndst = lax.rsqrt(jnp.maximum(ddst_ref[...], 1.0))
    nsrc = lax.rsqrt(jnp.maximum(dsrc_ref[...], 1.0))
    agg = (p_ref[0] + p_ref[1]) * ndst + b_ref[...]
    hmid = jnp.maximum(agg, 0.0)
    mm = jnp.dot(hmid, w_ref[...],
                 preferred_element_type=jnp.float32) * nsrc
    mu = jnp.mean(agg, axis=-1, keepdims=True)
    dvar = agg - mu
    var = jnp.mean(dvar * dvar, axis=-1, keepdims=True)
    ln = dvar * lax.rsqrt(var + 1e-5) * g_ref[...] + bt_ref[...]
    o_ref[...] = jnp.where(flag_ref[0, 0] > 0.5, mm, ln)


_row_spec = pl.BlockSpec((BR, D), lambda i: (i, 0))
_col_spec = pl.BlockSpec((BR, 1), lambda i: (i, 0))
_full_spec = pl.BlockSpec((1, D), lambda i: (0, 0))
_w_spec = pl.BlockSpec((D, D), lambda i: (0, 0))
_p_spec = pl.BlockSpec((NC, BR, D), lambda i: (0, i, 0))
_one_spec = pl.BlockSpec((1, 1), lambda i: (0, 0))
_out_shape = jax.ShapeDtypeStruct((N, D), jnp.float32)

_mm1 = pl.pallas_call(
    _mm1_body, grid=(GRID,),
    in_specs=[_row_spec, _w_spec, _col_spec],
    out_specs=_row_spec, out_shape=_out_shape)

_post = pl.pallas_call(
    _post_body, grid=(GRID,),
    in_specs=[_p_spec, _col_spec, _full_spec, _w_spec, _col_spec,
              _full_spec, _full_spec, _one_spec],
    out_specs=_row_spec, out_shape=_out_shape)


def kernel(x, edge_index, W1, b1, W2, b2, ln_gamma, ln_beta):
    src = edge_index[0]
    dst = edge_index[1]
    src16 = src.reshape(NW, DEG_ROWS, 16)
    dst16 = dst.reshape(NW, DEG_ROWS, 16)
    src80 = src.reshape(NW, CH, C)
    dst80 = dst.reshape(NW, CH, C)

    degs = _deg_kernel(src16, dst16)[:, 0]
    dsrc = (degs[0, :N] + degs[1, :N]).reshape(N, 1)
    ddst = (degs[0, NPAD:NPAD + N] + degs[1, NPAD:NPAD + N]).reshape(N, 1)

    b1r = b1.reshape(1, D)
    b2r = b2.reshape(1, D)
    gr = ln_gamma.reshape(1, D)
    btr = ln_beta.reshape(1, D)

    h1 = _mm1(x, W1, dsrc)

    w_st = jnp.stack([W2, W2])
    b_st = jnp.stack([b1r, b2r])
    f_st = jnp.array([[[1.0]], [[0.0]]], jnp.float32)

    def body(h, xs):
        w_i, b_i, f_i = xs
        p = _edge_kernel(h, src80, dst80)
        h_next = _post(p, ddst, b_i, w_i, dsrc, gr, btr, f_i)
        return h_next, None

    y, _ = lax.scan(body, h1, (w_st, b_st, f_st))
    return y
